# Initial kernel scaffold; baseline (speedup 1.0000x reference)
import jax
import jax.numpy as jnp
from jax.experimental import pallas as pl


def kernel(h, c, edge_index, U_iou_w, U_f_w, U_f_b, b_iou):
    raise NotImplementedError("write your pallas kernel here")


# speedup vs baseline: 6.0716x; 6.0716x over previous
import functools

import jax
import jax.numpy as jnp
from jax import lax
from jax.experimental import pallas as pl
from jax.experimental.pallas import tpu as pltpu
from jax.experimental.pallas import tpu_sc as plsc

N = 10000
E = 320000
H = 128

NC = 2
NT = 16
K = 80
NE_T = E // NT
NB = NE_T // K
NQ = 10
NBC = NB // NQ
NP = 10240
RPT = NP // NT

ROW_BLK = 2000


def _pre_body(h_ref, c_ref, wt_ref, b_ref, p_ref):
    g = jax.nn.sigmoid(
        jnp.dot(h_ref[...], wt_ref[...], preferred_element_type=jnp.float32)
        + b_ref[...])
    p_ref[...] = g * c_ref[...]


_pre = pl.pallas_call(
    _pre_body,
    grid=(N // ROW_BLK,),
    in_specs=[
        pl.BlockSpec((ROW_BLK, H), lambda i: (i, 0)),
        pl.BlockSpec((ROW_BLK, H), lambda i: (i, 0)),
        pl.BlockSpec((H, H), lambda i: (0, 0)),
        pl.BlockSpec((1, H), lambda i: (0, 0)),
    ],
    out_specs=pl.BlockSpec((ROW_BLK, H), lambda i: (i, 0)),
    out_shape=jax.ShapeDtypeStruct((N, H), jnp.float32),
)


def _post_body(ht_ref, ca_ref, wt_ref, b_ref, h_ref, c_ref):
    iou = (jnp.dot(ht_ref[...], wt_ref[...], preferred_element_type=jnp.float32)
           + b_ref[...])
    i = jax.nn.sigmoid(iou[:, :H])
    o = jax.nn.sigmoid(iou[:, H:2 * H])
    u = jnp.tanh(iou[:, 2 * H:])
    c_new = i * u + ca_ref[...]
    h_ref[...] = o * jnp.tanh(c_new)
    c_ref[...] = c_new


_post = pl.pallas_call(
    _post_body,
    grid=(N // ROW_BLK,),
    in_specs=[
        pl.BlockSpec((ROW_BLK, H), lambda i: (i, 0)),
        pl.BlockSpec((ROW_BLK, H), lambda i: (i, 0)),
        pl.BlockSpec((H, 3 * H), lambda i: (0, 0)),
        pl.BlockSpec((1, 3 * H), lambda i: (0, 0)),
    ],
    out_specs=[
        pl.BlockSpec((ROW_BLK, H), lambda i: (i, 0)),
        pl.BlockSpec((ROW_BLK, H), lambda i: (i, 0)),
    ],
    out_shape=[
        jax.ShapeDtypeStruct((N, H), jnp.float32),
        jax.ShapeDtypeStruct((N, H), jnp.float32),
    ],
)


def _edge_body(tab, src3, dst3, out, src_v, dst_v, rows_v, acc, sem):
    c = lax.axis_index("c")
    s = lax.axis_index("s")

    zero16 = jnp.zeros((16,), jnp.float32)

    def _zrow(i, carry):
        for j in range(H // 16):
            rows_v[i, 16 * j:16 * (j + 1)] = zero16
        return carry

    lax.fori_loop(0, K, _zrow, 0)
    base = s * RPT
    for t in range(RPT // K):
        pltpu.sync_copy(rows_v, acc.at[pl.ds(base + K * t, K)])
    plsc.subcore_barrier()

    def _chunk(q, carry):
        pltpu.sync_copy(src3.at[c, s, q], src_v)
        pltpu.sync_copy(dst3.at[s, q], dst_v)

        def _blk(j, carry2):
            pltpu.async_copy(tab.at[src_v.at[j]], rows_v, sem).wait()
            pltpu.sync_copy(rows_v, acc.at[dst_v.at[j]], add=True)
            return carry2

        lax.fori_loop(0, NBC, _blk, 0)
        return carry

    lax.fori_loop(0, NQ, _chunk, 0)
    plsc.subcore_barrier()

    for t in range(RPT // K):
        pltpu.sync_copy(acc.at[pl.ds(base + K * t, K)], rows_v)
        pltpu.sync_copy(rows_v, out.at[c, pl.ds(base + K * t, K)])


@functools.lru_cache(maxsize=1)
def _edge_kernel():
    return pl.kernel(
        _edge_body,
        out_type=pltpu.HBM((NC, NP, H), jnp.float32),
        mesh=plsc.VectorSubcoreMesh(core_axis_name="c", subcore_axis_name="s"),
        scratch_types=[
            pltpu.VMEM((NBC, K), jnp.int32),
            pltpu.VMEM((NBC, K), jnp.int32),
            pltpu.VMEM((K, H), jnp.float32),
            pltpu.VMEM_SHARED((NP, H), jnp.float32),
            pltpu.SemaphoreType.DMA,
        ],
    )


@jax.jit
def kernel(h, c, edge_index, U_iou_w, U_f_w, U_f_b, b_iou):
    src = edge_index[0].astype(jnp.int32)
    dst = edge_index[1].astype(jnp.int32)

    p = _pre(h, c, U_f_w.T, U_f_b.reshape(1, H))

    tab = jnp.concatenate([h, p], axis=0)
    src3 = jnp.stack([src, src + N]).reshape(NC, NT, NQ, NBC, K)
    dst3 = dst.reshape(NT, NQ, NBC, K)

    agg = _edge_kernel()(tab, src3, dst3)
    h_new, c_new = _post(agg[0, :N], agg[1, :N], U_iou_w.T, b_iou)
    return h_new, c_new
